# baseline (speedup 1.0000x reference)
import functools

import jax
import jax.numpy as jnp
from jax import lax
from jax.experimental import pallas as pl
from jax.experimental.pallas import tpu as pltpu
from jax.experimental.pallas import tpu_sc as plsc

N_PAD = 10240
ROW_BLK = 512
E_PAD = 163840
EDGES_PER_TILE = E_PAD // 32


def _dense_body(x_type_ref, x_sent_ref, wt_type_ref, wt_sent_ref,
                a_src_ref, a_dst_ref, h_type_ref, t_ref, d_ref, c_ref,
                acc_ref):
    i = pl.program_id(0)
    nsteps = pl.num_programs(0)
    xt = x_type_ref[...]
    xs = x_sent_ref[...]
    wt = wt_type_ref[...]
    ws = wt_sent_ref[...]
    h_type = jnp.dot(xt, wt, preferred_element_type=jnp.float32)
    h_type_ref[...] = h_type
    t_blk = jnp.dot(h_type, a_src_ref[...], preferred_element_type=jnp.float32)
    t_ref[...] = t_blk
    vs = jnp.dot(ws, a_dst_ref[...], preferred_element_type=jnp.float32)
    d_blk = jnp.dot(xs, vs, preferred_element_type=jnp.float32)
    d_ref[...] = d_blk

    @pl.when(i == 0)
    def _():
        acc_ref[0] = -jnp.inf
        acc_ref[1] = -jnp.inf

    acc_ref[0] = jnp.maximum(acc_ref[0], jnp.max(t_blk))
    acc_ref[1] = jnp.maximum(acc_ref[1], jnp.max(d_blk))

    @pl.when(i == nsteps - 1)
    def _():
        m = acc_ref[0] + acc_ref[1]
        c_ref[0, 0] = jnp.maximum(m, 0.2 * m)


@functools.partial(jax.jit, static_argnames=())
def _dense_stage(x_type_p, x_sent_p, wt_type, wt_sent, a_src128, a_dst128):
    nblk = N_PAD // ROW_BLK
    return pl.pallas_call(
        _dense_body,
        grid=(nblk,),
        in_specs=[
            pl.BlockSpec((ROW_BLK, 512), lambda i: (i, 0)),
            pl.BlockSpec((ROW_BLK, 512), lambda i: (i, 0)),
            pl.BlockSpec((512, 768), lambda i: (0, 0)),
            pl.BlockSpec((512, 768), lambda i: (0, 0)),
            pl.BlockSpec((768, 128), lambda i: (0, 0)),
            pl.BlockSpec((768, 128), lambda i: (0, 0)),
        ],
        out_specs=[
            pl.BlockSpec((ROW_BLK, 768), lambda i: (i, 0)),
            pl.BlockSpec((ROW_BLK, 128), lambda i: (i, 0)),
            pl.BlockSpec((ROW_BLK, 128), lambda i: (i, 0)),
            pl.BlockSpec(memory_space=pltpu.SMEM),
        ],
        out_shape=[
            jax.ShapeDtypeStruct((N_PAD, 768), jnp.float32),
            jax.ShapeDtypeStruct((N_PAD, 128), jnp.float32),
            jax.ShapeDtypeStruct((N_PAD, 128), jnp.float32),
            jax.ShapeDtypeStruct((1, 1), jnp.float32),
        ],
        scratch_shapes=[pltpu.SMEM((2,), jnp.float32)],
    )(x_type_p, x_sent_p, wt_type, wt_sent, a_src128, a_dst128)


_SC_MESH = plsc.VectorSubcoreMesh(core_axis_name="c", subcore_axis_name="s")


@functools.partial(
    pl.kernel,
    out_type=[
        jax.ShapeDtypeStruct((E_PAD,), jnp.float32),
        jax.ShapeDtypeStruct((32, N_PAD), jnp.float32),
    ],
    mesh=_SC_MESH,
    compiler_params=pltpu.CompilerParams(needs_layout_passes=False, use_tc_tiling_on_sc=False),
    scratch_types=[
        pltpu.VMEM((N_PAD,), jnp.float32),
        pltpu.VMEM((N_PAD,), jnp.float32),
        pltpu.VMEM((16,), jnp.float32),
        pltpu.VMEM((EDGES_PER_TILE,), jnp.int32),
        pltpu.VMEM((EDGES_PER_TILE,), jnp.int32),
        pltpu.VMEM((EDGES_PER_TILE,), jnp.float32),
        pltpu.VMEM((N_PAD,), jnp.float32),
    ],
)
def _edge_scalar_stage(t_hbm, d_hbm, c_hbm, src_hbm, dst_hbm,
                       ex_hbm, sall_hbm,
                       t_v, d_v, c_v, src_v, dst_v, ex_v, s_v):
    wid = lax.axis_index("s") * 2 + lax.axis_index("c")
    base = wid * EDGES_PER_TILE
    pltpu.sync_copy(t_hbm, t_v)
    pltpu.sync_copy(d_hbm, d_v)
    pltpu.sync_copy(c_hbm, c_v)
    pltpu.sync_copy(src_hbm.at[pl.ds(base, EDGES_PER_TILE)], src_v)
    pltpu.sync_copy(dst_hbm.at[pl.ds(base, EDGES_PER_TILE)], dst_v)
    zero16 = jnp.zeros((16,), jnp.float32)

    def _zero(i, carry):
        s_v[pl.ds(i * 16, 16)] = zero16
        return carry

    lax.fori_loop(0, N_PAD // 16, _zero, 0)
    cvec = c_v[...]

    def _edges(j, carry):
        sl = pl.ds(j * 16, 16)
        s16 = src_v[sl]
        d16 = dst_v[sl]
        tg = plsc.load_gather(t_v, [s16])
        dg = plsc.load_gather(d_v, [d16])
        x = tg + dg
        e = jnp.maximum(x, 0.2 * x)
        exv = jnp.exp(e - cvec)
        ex_v[sl] = exv
        plsc.addupdate_scatter(s_v, [d16], exv)
        return carry

    lax.fori_loop(0, EDGES_PER_TILE // 16, _edges, 0)
    pltpu.sync_copy(ex_v, ex_hbm.at[pl.ds(base, EDGES_PER_TILE)])
    pltpu.sync_copy(s_v, sall_hbm.at[wid])


_COLS_PER_TILE = N_PAD // 32


@functools.partial(
    pl.kernel,
    out_type=jax.ShapeDtypeStruct((N_PAD,), jnp.float32),
    mesh=_SC_MESH,
    compiler_params=pltpu.CompilerParams(needs_layout_passes=False, use_tc_tiling_on_sc=False),
    scratch_types=[
        pltpu.VMEM((32, _COLS_PER_TILE), jnp.float32),
        pltpu.VMEM((_COLS_PER_TILE,), jnp.float32),
    ],
)
def _combine_stage(sall_hbm, rec_hbm, buf_v, acc_v):
    wid = lax.axis_index("s") * 2 + lax.axis_index("c")
    col0 = wid * _COLS_PER_TILE
    pltpu.sync_copy(sall_hbm.at[:, pl.ds(col0, _COLS_PER_TILE)], buf_v)
    nchunk = _COLS_PER_TILE // 16
    zero16 = jnp.zeros((16,), jnp.float32)

    def _zero(i, carry):
        acc_v[pl.ds(i * 16, 16)] = zero16
        return carry

    lax.fori_loop(0, nchunk, _zero, 0)

    def _add_row(c, carry):
        for k in range(nchunk):
            sl = pl.ds(k * 16, 16)
            acc_v[sl] = acc_v[sl] + buf_v[c, sl]
        return carry

    lax.fori_loop(0, 32, _add_row, 0)
    for k in range(nchunk):
        sl = pl.ds(k * 16, 16)
        acc_v[sl] = 1.0 / (acc_v[sl] + 1e-9)
    pltpu.sync_copy(acc_v, rec_hbm.at[pl.ds(col0, _COLS_PER_TILE)])


_Q = N_PAD // 8
_ACC_ROWS = _Q + 16
_BATCH = 64
_CHUNK = 1024
_EPT_D = E_PAD // 16


@functools.partial(
    pl.kernel,
    out_type=jax.ShapeDtypeStruct((N_PAD, 768), jnp.float32),
    mesh=_SC_MESH,
    compiler_params=pltpu.CompilerParams(needs_layout_passes=False, use_tc_tiling_on_sc=False),
    scratch_types=[
        pltpu.VMEM((N_PAD,), jnp.float32),
        pltpu.VMEM((_CHUNK,), jnp.int32),
        pltpu.VMEM((_CHUNK,), jnp.int32),
        pltpu.VMEM((_CHUNK,), jnp.float32),
        pltpu.VMEM((_CHUNK + _BATCH,), jnp.int32),
        pltpu.VMEM((_CHUNK + _BATCH,), jnp.int32),
        pltpu.VMEM((_CHUNK + _BATCH,), jnp.float32),
        pltpu.VMEM((_BATCH,), jnp.int32),
        pltpu.VMEM((_BATCH,), jnp.int32),
        pltpu.VMEM((_BATCH,), jnp.float32),
        pltpu.VMEM((_BATCH, 768), jnp.float32),
        pltpu.SemaphoreType.DMA,
        pltpu.VMEM_SHARED((_ACC_ROWS, 768), jnp.float32),
    ],
)
def _aggregate_stage(h_hbm, src_hbm, dst_hbm, ex_hbm, rec_hbm, out_hbm,
                     rec_v, srcc, dstc, exc, csrc, cdst, calpha,
                     bsrc, bdst, balpha, rowbuf, sem, acc_sh):
    core = lax.axis_index("c")
    sub = lax.axis_index("s")
    pltpu.sync_copy(rec_hbm, rec_v)
    zero16f = jnp.zeros((16,), jnp.float32)
    zero16i = jnp.zeros((16,), jnp.int32)
    junk16 = jnp.full((16,), _Q, jnp.int32)
    ebase = sub * _EPT_D

    for r in range(4):
        q = 4 * core + r
        base = q * _Q

        def _zrow(i, carry):
            for k in range(48):
                rowbuf[i, pl.ds(k * 16, 16)] = zero16f
            return carry

        lax.fori_loop(0, _BATCH, _zrow, 0)
        r0 = sub * (_ACC_ROWS // 16)
        pltpu.sync_copy(rowbuf, acc_sh.at[pl.ds(r0, 64)])
        pltpu.sync_copy(rowbuf.at[pl.ds(0, 17)], acc_sh.at[pl.ds(r0 + 64, 17)])
        plsc.subcore_barrier()

        def _chunk(ch, carry):
            e0 = ebase + ch * _CHUNK
            pltpu.sync_copy(src_hbm.at[pl.ds(e0, _CHUNK)], srcc)
            pltpu.sync_copy(dst_hbm.at[pl.ds(e0, _CHUNK)], dstc)
            pltpu.sync_copy(ex_hbm.at[pl.ds(e0, _CHUNK)], exc)

            def _grp(g, off):
                sl = pl.ds(g * 16, 16)
                d16 = dstc[sl]
                rel = d16 - base
                m = (rel >= 0) & (rel < _Q)
                rg = plsc.load_gather(rec_v, [d16])
                a16 = exc[sl] * rg
                plsc.store_compressed(csrc.at[pl.ds(off, 16)], srcc[sl], mask=m)
                plsc.store_compressed(cdst.at[pl.ds(off, 16)], rel, mask=m)
                plsc.store_compressed(calpha.at[pl.ds(off, 16)], a16, mask=m)
                return off + jnp.sum(m.astype(jnp.int32))

            n_sel = lax.fori_loop(0, _CHUNK // 16, _grp, 0)
            for k in range(_BATCH // 16):
                sl = pl.ds(n_sel + k * 16, 16)
                csrc[sl] = zero16i
                cdst[sl] = junk16
                calpha[sl] = zero16f
            nb = (n_sel + _BATCH - 1) // _BATCH

            def _batch(b, carry2):
                off0 = b * _BATCH
                for k in range(_BATCH // 16):
                    sl = pl.ds(k * 16, 16)
                    slc = pl.ds(off0 + k * 16, 16)
                    bsrc[sl] = csrc[slc]
                    bdst[sl] = cdst[slc]
                    balpha[sl] = calpha[slc]
                pltpu.async_copy(h_hbm.at[bsrc], rowbuf, sem).wait()

                def _srow(i, carry3):
                    av = plsc.load_gather(balpha, [zero16i + i])
                    for k in range(48):
                        sl2 = pl.ds(k * 16, 16)
                        rowbuf[i, sl2] = rowbuf[i, sl2] * av
                    return carry3

                lax.fori_loop(0, _BATCH, _srow, 0)
                pltpu.sync_copy(rowbuf, acc_sh.at[bdst], add=True)
                return carry2

            lax.fori_loop(0, nb, _batch, 0)
            return carry

        lax.fori_loop(0, _EPT_D // _CHUNK, _chunk, 0)
        plsc.subcore_barrier()

        w0 = sub * (_Q // 16)
        for part0, sz in ((0, 64), (64, 16)):
            pltpu.sync_copy(acc_sh.at[pl.ds(w0 + part0, sz)],
                            rowbuf.at[pl.ds(0, sz)])

            def _erow(i, carry):
                for k in range(48):
                    sl2 = pl.ds(k * 16, 16)
                    x = rowbuf[i, sl2]
                    rowbuf[i, sl2] = jnp.where(
                        x > 0.0, x, jnp.exp(jnp.minimum(x, 0.0)) - 1.0)
                return carry

            lax.fori_loop(0, sz, _erow, 0)
            pltpu.sync_copy(rowbuf.at[pl.ds(0, sz)],
                            out_hbm.at[pl.ds(base + w0 + part0, sz)])
        plsc.subcore_barrier()


def kernel(x_sent, x_type, W_sent, W_type, a_src, a_dst, edge_index):
    n_sent, _ = x_sent.shape
    n_type, _ = x_type.shape
    x_sent_p = jnp.pad(x_sent, ((0, N_PAD - n_sent), (0, 0)))
    x_type_p = jnp.pad(x_type, ((0, N_PAD - n_type), (0, 0)))
    a_src128 = jnp.broadcast_to(a_src[:, None], (768, 128))
    a_dst128 = jnp.broadcast_to(a_dst[:, None], (768, 128))

    h_type_p, t128, d128, c11 = _dense_stage(
        x_type_p, x_sent_p, W_type.T, W_sent.T, a_src128, a_dst128)
    t = t128[:, 0]
    d = d128[:, 0]
    C = c11[0, 0]

    src = edge_index[0]
    dst = edge_index[1]
    n_extra = E_PAD - src.shape[0]
    src_p = jnp.concatenate([src, jnp.zeros((n_extra,), jnp.int32)])
    dst_p = jnp.concatenate([dst, jnp.full((n_extra,), n_sent, jnp.int32)])
    c16 = jnp.broadcast_to(C, (16,))

    ex_p, s_all = _edge_scalar_stage(t, d, c16, src_p, dst_p)
    rec = _combine_stage(s_all)
    out_p = _aggregate_stage(h_type_p, src_p, dst_p, ex_p, rec)
    return out_p[:n_sent]


# speedup vs baseline: 1.5002x; 1.5002x over previous
import functools

import jax
import jax.numpy as jnp
from jax import lax
from jax.experimental import pallas as pl
from jax.experimental.pallas import tpu as pltpu
from jax.experimental.pallas import tpu_sc as plsc

N_PAD = 10240
ROW_BLK = 512
E_PAD = 163840
EDGES_PER_TILE = E_PAD // 32


def _dense_body(x_type_ref, x_sent_ref, wt_type_ref, wt_sent_ref,
                a_src_ref, a_dst_ref, h_type_ref, t_ref, d_ref, c_ref,
                acc_ref):
    i = pl.program_id(0)
    nsteps = pl.num_programs(0)
    xt = x_type_ref[...]
    xs = x_sent_ref[...]
    wt = wt_type_ref[...]
    ws = wt_sent_ref[...]
    h_type = jnp.dot(xt, wt, preferred_element_type=jnp.float32)
    h_type_ref[...] = h_type
    t_blk = jnp.dot(h_type, a_src_ref[...], preferred_element_type=jnp.float32)
    t_ref[...] = t_blk
    vs = jnp.dot(ws, a_dst_ref[...], preferred_element_type=jnp.float32)
    d_blk = jnp.dot(xs, vs, preferred_element_type=jnp.float32)
    d_ref[...] = d_blk

    @pl.when(i == 0)
    def _():
        acc_ref[0] = -jnp.inf
        acc_ref[1] = -jnp.inf

    acc_ref[0] = jnp.maximum(acc_ref[0], jnp.max(t_blk))
    acc_ref[1] = jnp.maximum(acc_ref[1], jnp.max(d_blk))

    @pl.when(i == nsteps - 1)
    def _():
        m = acc_ref[0] + acc_ref[1]
        c_ref[0, 0] = jnp.maximum(m, 0.2 * m)


@functools.partial(jax.jit, static_argnames=())
def _dense_stage(x_type_p, x_sent_p, wt_type, wt_sent, a_src128, a_dst128):
    nblk = N_PAD // ROW_BLK
    return pl.pallas_call(
        _dense_body,
        grid=(nblk,),
        in_specs=[
            pl.BlockSpec((ROW_BLK, 512), lambda i: (i, 0)),
            pl.BlockSpec((ROW_BLK, 512), lambda i: (i, 0)),
            pl.BlockSpec((512, 768), lambda i: (0, 0)),
            pl.BlockSpec((512, 768), lambda i: (0, 0)),
            pl.BlockSpec((768, 128), lambda i: (0, 0)),
            pl.BlockSpec((768, 128), lambda i: (0, 0)),
        ],
        out_specs=[
            pl.BlockSpec((ROW_BLK, 768), lambda i: (i, 0)),
            pl.BlockSpec((ROW_BLK, 128), lambda i: (i, 0)),
            pl.BlockSpec((ROW_BLK, 128), lambda i: (i, 0)),
            pl.BlockSpec(memory_space=pltpu.SMEM),
        ],
        out_shape=[
            jax.ShapeDtypeStruct((N_PAD, 768), jnp.float32),
            jax.ShapeDtypeStruct((N_PAD, 128), jnp.float32),
            jax.ShapeDtypeStruct((N_PAD, 128), jnp.float32),
            jax.ShapeDtypeStruct((1, 1), jnp.float32),
        ],
        scratch_shapes=[pltpu.SMEM((2,), jnp.float32)],
    )(x_type_p, x_sent_p, wt_type, wt_sent, a_src128, a_dst128)


_SC_MESH = plsc.VectorSubcoreMesh(core_axis_name="c", subcore_axis_name="s")


@functools.partial(
    pl.kernel,
    out_type=[
        jax.ShapeDtypeStruct((E_PAD,), jnp.float32),
        jax.ShapeDtypeStruct((32, N_PAD), jnp.float32),
    ],
    mesh=_SC_MESH,
    compiler_params=pltpu.CompilerParams(needs_layout_passes=False, use_tc_tiling_on_sc=False),
    scratch_types=[
        pltpu.VMEM((N_PAD,), jnp.float32),
        pltpu.VMEM((N_PAD,), jnp.float32),
        pltpu.VMEM((16,), jnp.float32),
        pltpu.VMEM((EDGES_PER_TILE,), jnp.int32),
        pltpu.VMEM((EDGES_PER_TILE,), jnp.int32),
        pltpu.VMEM((EDGES_PER_TILE,), jnp.float32),
        pltpu.VMEM((N_PAD,), jnp.float32),
    ],
)
def _edge_scalar_stage(t_hbm, d_hbm, c_hbm, src_hbm, dst_hbm,
                       ex_hbm, sall_hbm,
                       t_v, d_v, c_v, src_v, dst_v, ex_v, s_v):
    wid = lax.axis_index("s") * 2 + lax.axis_index("c")
    base = wid * EDGES_PER_TILE
    pltpu.sync_copy(t_hbm, t_v)
    pltpu.sync_copy(d_hbm, d_v)
    pltpu.sync_copy(c_hbm, c_v)
    pltpu.sync_copy(src_hbm.at[pl.ds(base, EDGES_PER_TILE)], src_v)
    pltpu.sync_copy(dst_hbm.at[pl.ds(base, EDGES_PER_TILE)], dst_v)
    zero16 = jnp.zeros((16,), jnp.float32)

    def _zero(i, carry):
        s_v[pl.ds(i * 16, 16)] = zero16
        return carry

    lax.fori_loop(0, N_PAD // 16, _zero, 0)
    cvec = c_v[...]

    def _edges(j, carry):
        sl = pl.ds(j * 16, 16)
        s16 = src_v[sl]
        d16 = dst_v[sl]
        tg = plsc.load_gather(t_v, [s16])
        dg = plsc.load_gather(d_v, [d16])
        x = tg + dg
        e = jnp.maximum(x, 0.2 * x)
        exv = jnp.exp(e - cvec)
        ex_v[sl] = exv
        plsc.addupdate_scatter(s_v, [d16], exv)
        return carry

    lax.fori_loop(0, EDGES_PER_TILE // 16, _edges, 0)
    pltpu.sync_copy(ex_v, ex_hbm.at[pl.ds(base, EDGES_PER_TILE)])
    pltpu.sync_copy(s_v, sall_hbm.at[wid])


_COLS_PER_TILE = N_PAD // 32


@functools.partial(
    pl.kernel,
    out_type=jax.ShapeDtypeStruct((N_PAD,), jnp.float32),
    mesh=_SC_MESH,
    compiler_params=pltpu.CompilerParams(needs_layout_passes=False, use_tc_tiling_on_sc=False),
    scratch_types=[
        pltpu.VMEM((32, _COLS_PER_TILE), jnp.float32),
        pltpu.VMEM((_COLS_PER_TILE,), jnp.float32),
    ],
)
def _combine_stage(sall_hbm, rec_hbm, buf_v, acc_v):
    wid = lax.axis_index("s") * 2 + lax.axis_index("c")
    col0 = wid * _COLS_PER_TILE
    pltpu.sync_copy(sall_hbm.at[:, pl.ds(col0, _COLS_PER_TILE)], buf_v)
    nchunk = _COLS_PER_TILE // 16
    zero16 = jnp.zeros((16,), jnp.float32)

    def _zero(i, carry):
        acc_v[pl.ds(i * 16, 16)] = zero16
        return carry

    lax.fori_loop(0, nchunk, _zero, 0)

    def _add_row(c, carry):
        for k in range(nchunk):
            sl = pl.ds(k * 16, 16)
            acc_v[sl] = acc_v[sl] + buf_v[c, sl]
        return carry

    lax.fori_loop(0, 32, _add_row, 0)
    for k in range(nchunk):
        sl = pl.ds(k * 16, 16)
        acc_v[sl] = 1.0 / (acc_v[sl] + 1e-9)
    pltpu.sync_copy(acc_v, rec_hbm.at[pl.ds(col0, _COLS_PER_TILE)])


_Q = 640
_ACC_ROWS = _Q + 16
_BATCH = 32
_RING = 3
_CHUNK = 2048
_EPT_D = E_PAD // 16
_CSIZE = 2080


@functools.partial(
    pl.kernel,
    out_type=jax.ShapeDtypeStruct((N_PAD, 768), jnp.float32),
    mesh=_SC_MESH,
    compiler_params=pltpu.CompilerParams(needs_layout_passes=False, use_tc_tiling_on_sc=False),
    scratch_types=[
        pltpu.VMEM((_CHUNK,), jnp.int32),
        pltpu.VMEM((_CHUNK,), jnp.int32),
        pltpu.VMEM((_CHUNK,), jnp.float32),
        pltpu.VMEM((_CSIZE,), jnp.int32),
        pltpu.VMEM((_CSIZE,), jnp.int32),
        pltpu.VMEM((_CSIZE,), jnp.float32),
        [pltpu.VMEM((_BATCH,), jnp.int32) for _ in range(_RING)],
        [pltpu.VMEM((_BATCH,), jnp.int32) for _ in range(_RING)],
        [pltpu.VMEM((_BATCH,), jnp.float32) for _ in range(_RING)],
        [pltpu.VMEM((_BATCH, 768), jnp.float32) for _ in range(_RING)],
        pltpu.VMEM((16,), jnp.float32),
        [pltpu.SemaphoreType.DMA for _ in range(_RING)],
        [pltpu.SemaphoreType.DMA for _ in range(_RING)],
        pltpu.VMEM_SHARED((_ACC_ROWS, 768), jnp.float32),
    ],
)
def _aggregate_stage(h_hbm, src_hbm, dst_hbm, ex_hbm, rec_hbm, out_hbm,
                     srcc, dstc, exc, csrc, cdst, cex,
                     bsrc, bdst, bex, rowbuf, recb, gsem, ssem, acc_sh):
    core = lax.axis_index("c")
    sub = lax.axis_index("s")
    zero16f = jnp.zeros((16,), jnp.float32)
    zero16i = jnp.zeros((16,), jnp.int32)
    junk16 = jnp.full((16,), _Q, jnp.int32)
    ebase = sub * _EPT_D

    def _prep(p, b):
        for k in range(_BATCH // 16):
            sl = pl.ds(k * 16, 16)
            slc = pl.ds(b * _BATCH + k * 16, 16)
            bsrc[p][sl] = csrc[slc]
            bdst[p][sl] = cdst[slc]
            bex[p][sl] = cex[slc]

    def _scale(p):
        def _srow(i, carry):
            av = plsc.load_gather(bex[p], [zero16i + i])
            for k in range(48):
                sl2 = pl.ds(k * 16, 16)
                rowbuf[p][i, sl2] = rowbuf[p][i, sl2] * av
            return carry

        lax.fori_loop(0, _BATCH, _srow, 0)

    def _round(r, rcarry):
        sector = 8 * core + r
        base = sector * _Q

        def _zrow(i, carry):
            for k in range(48):
                rowbuf[0][i, pl.ds(k * 16, 16)] = zero16f
            return carry

        lax.fori_loop(0, _BATCH, _zrow, 0)
        r0 = sub * (_ACC_ROWS // 16)
        pltpu.sync_copy(rowbuf[0], acc_sh.at[pl.ds(r0, 32)])
        pltpu.sync_copy(rowbuf[0].at[pl.ds(0, 9)],
                        acc_sh.at[pl.ds(r0 + 32, 9)])
        plsc.subcore_barrier()

        def _chunk(ch, off_in):
            e0 = ebase + ch * _CHUNK
            pltpu.sync_copy(src_hbm.at[pl.ds(e0, _CHUNK)], srcc)
            pltpu.sync_copy(dst_hbm.at[pl.ds(e0, _CHUNK)], dstc)
            pltpu.sync_copy(ex_hbm.at[pl.ds(e0, _CHUNK)], exc)

            def _grp(g, off):
                sl = pl.ds(g * 16, 16)
                rel = dstc[sl] - base
                m = (rel >= 0) & (rel < _Q)
                plsc.store_compressed(csrc.at[pl.ds(off, 16)], srcc[sl], mask=m)
                plsc.store_compressed(cdst.at[pl.ds(off, 16)], rel, mask=m)
                plsc.store_compressed(cex.at[pl.ds(off, 16)], exc[sl], mask=m)
                return off + jnp.sum(m.astype(jnp.int32))

            return lax.fori_loop(0, _CHUNK // 16, _grp, off_in)

        n_sel = lax.fori_loop(0, _EPT_D // _CHUNK, _chunk, 0)
        for k in range(224 // 16):
            sl = pl.ds(n_sel + k * 16, 16)
            csrc[sl] = zero16i
            cdst[sl] = junk16
            cex[sl] = zero16f

        for p in range(_RING):
            _prep(p, p)
            pltpu.async_copy(h_hbm.at[bsrc[p]], rowbuf[p], gsem[p])
        nb_total = (n_sel + _BATCH - 1) // _BATCH
        nbt = jnp.maximum((nb_total + _RING - 1) // _RING, 1)

        def _iter(t, carry):
            for p in range(_RING):
                pltpu.make_async_copy(h_hbm.at[bsrc[p]], rowbuf[p],
                                      gsem[p]).wait()
                _scale(p)
                pltpu.async_copy(rowbuf[p], acc_sh.at[bdst[p]], ssem[p],
                                 add=True)
            for p in range(_RING):
                pltpu.make_async_copy(rowbuf[p], acc_sh.at[bdst[p]],
                                      ssem[p]).wait()
                _prep(p, t * _RING + _RING + p)
                pltpu.async_copy(h_hbm.at[bsrc[p]], rowbuf[p], gsem[p])
            return carry

        lax.fori_loop(0, nbt, _iter, 0)
        for p in range(_RING):
            pltpu.make_async_copy(h_hbm.at[bsrc[p]], rowbuf[p], gsem[p]).wait()
        plsc.subcore_barrier()

        w0 = sub * (_Q // 16)
        for part0 in (0, 16, 24):
            gr0 = base + w0 + part0
            pltpu.sync_copy(acc_sh.at[pl.ds(w0 + part0, 16)],
                            rowbuf[0].at[pl.ds(0, 16)])
            pltpu.sync_copy(rec_hbm.at[pl.ds(gr0, 16)], recb)

            def _erow(i, carry):
                rv = plsc.load_gather(recb, [zero16i + i])
                for k in range(48):
                    sl2 = pl.ds(k * 16, 16)
                    x = rowbuf[0][i, sl2] * rv
                    rowbuf[0][i, sl2] = jnp.where(
                        x > 0.0, x, jnp.exp(jnp.minimum(x, 0.0)) - 1.0)
                return carry

            lax.fori_loop(0, 16, _erow, 0)
            pltpu.sync_copy(rowbuf[0].at[pl.ds(0, 16)],
                            out_hbm.at[pl.ds(gr0, 16)])
        plsc.subcore_barrier()
        return rcarry

    lax.fori_loop(0, 8, _round, 0)


def kernel(x_sent, x_type, W_sent, W_type, a_src, a_dst, edge_index):
    n_sent, _ = x_sent.shape
    n_type, _ = x_type.shape
    x_sent_p = jnp.pad(x_sent, ((0, N_PAD - n_sent), (0, 0)))
    x_type_p = jnp.pad(x_type, ((0, N_PAD - n_type), (0, 0)))
    a_src128 = jnp.broadcast_to(a_src[:, None], (768, 128))
    a_dst128 = jnp.broadcast_to(a_dst[:, None], (768, 128))

    h_type_p, t128, d128, c11 = _dense_stage(
        x_type_p, x_sent_p, W_type.T, W_sent.T, a_src128, a_dst128)
    t = t128[:, 0]
    d = d128[:, 0]
    C = c11[0, 0]

    src = edge_index[0]
    dst = edge_index[1]
    n_e = src.shape[0]
    per_tile_pad = (E_PAD - n_e) // 16
    src_p = jnp.concatenate(
        [src.reshape(16, n_e // 16),
         jnp.zeros((16, per_tile_pad), jnp.int32)], axis=1).reshape(-1)
    dst_p = jnp.concatenate(
        [dst.reshape(16, n_e // 16),
         jnp.full((16, per_tile_pad), n_sent, jnp.int32)], axis=1).reshape(-1)
    c16 = jnp.broadcast_to(C, (16,))

    ex_p, s_all = _edge_scalar_stage(t, d, c16, src_p, dst_p)
    rec = _combine_stage(s_all)
    out_p = _aggregate_stage(h_type_p, src_p, dst_p, ex_p, rec)
    return out_p[:n_sent]


# speedup vs baseline: 1.5447x; 1.0297x over previous
import functools

import jax
import jax.numpy as jnp
from jax import lax
from jax.experimental import pallas as pl
from jax.experimental.pallas import tpu as pltpu
from jax.experimental.pallas import tpu_sc as plsc

N_PAD = 10240
ROW_BLK = 512
E_PAD = 163840
EDGES_PER_TILE = E_PAD // 32


def _dense_body(x_type_ref, x_sent_ref, wt_type_ref, wt_sent_ref,
                a_src_ref, a_dst_ref, h_type_ref, t_ref, d_ref, c_ref,
                acc_ref):
    i = pl.program_id(0)
    nsteps = pl.num_programs(0)
    xt = x_type_ref[...]
    xs = x_sent_ref[...]
    wt = wt_type_ref[...]
    ws = wt_sent_ref[...]
    h_type = jnp.dot(xt, wt, preferred_element_type=jnp.float32)
    h_type_ref[...] = h_type
    t_blk = jnp.dot(h_type, a_src_ref[...], preferred_element_type=jnp.float32)
    t_ref[...] = t_blk
    vs = jnp.dot(ws, a_dst_ref[...], preferred_element_type=jnp.float32)
    d_blk = jnp.dot(xs, vs, preferred_element_type=jnp.float32)
    d_ref[...] = d_blk

    @pl.when(i == 0)
    def _():
        acc_ref[0] = -jnp.inf
        acc_ref[1] = -jnp.inf

    acc_ref[0] = jnp.maximum(acc_ref[0], jnp.max(t_blk))
    acc_ref[1] = jnp.maximum(acc_ref[1], jnp.max(d_blk))

    @pl.when(i == nsteps - 1)
    def _():
        m = acc_ref[0] + acc_ref[1]
        c_ref[0, 0] = jnp.maximum(m, 0.2 * m)


@functools.partial(jax.jit, static_argnames=())
def _dense_stage(x_type_p, x_sent_p, wt_type, wt_sent, a_src128, a_dst128):
    nblk = N_PAD // ROW_BLK
    return pl.pallas_call(
        _dense_body,
        grid=(nblk,),
        in_specs=[
            pl.BlockSpec((ROW_BLK, 512), lambda i: (i, 0)),
            pl.BlockSpec((ROW_BLK, 512), lambda i: (i, 0)),
            pl.BlockSpec((512, 768), lambda i: (0, 0)),
            pl.BlockSpec((512, 768), lambda i: (0, 0)),
            pl.BlockSpec((768, 128), lambda i: (0, 0)),
            pl.BlockSpec((768, 128), lambda i: (0, 0)),
        ],
        out_specs=[
            pl.BlockSpec((ROW_BLK, 768), lambda i: (i, 0)),
            pl.BlockSpec((ROW_BLK, 128), lambda i: (i, 0)),
            pl.BlockSpec((ROW_BLK, 128), lambda i: (i, 0)),
            pl.BlockSpec(memory_space=pltpu.SMEM),
        ],
        out_shape=[
            jax.ShapeDtypeStruct((N_PAD, 768), jnp.float32),
            jax.ShapeDtypeStruct((N_PAD, 128), jnp.float32),
            jax.ShapeDtypeStruct((N_PAD, 128), jnp.float32),
            jax.ShapeDtypeStruct((1, 1), jnp.float32),
        ],
        scratch_shapes=[pltpu.SMEM((2,), jnp.float32)],
    )(x_type_p, x_sent_p, wt_type, wt_sent, a_src128, a_dst128)


_SC_MESH = plsc.VectorSubcoreMesh(core_axis_name="c", subcore_axis_name="s")


@functools.partial(
    pl.kernel,
    out_type=[
        jax.ShapeDtypeStruct((E_PAD,), jnp.float32),
        jax.ShapeDtypeStruct((32, N_PAD), jnp.float32),
    ],
    mesh=_SC_MESH,
    compiler_params=pltpu.CompilerParams(needs_layout_passes=False, use_tc_tiling_on_sc=False),
    scratch_types=[
        pltpu.VMEM((N_PAD,), jnp.float32),
        pltpu.VMEM((N_PAD,), jnp.float32),
        pltpu.VMEM((16,), jnp.float32),
        pltpu.VMEM((EDGES_PER_TILE,), jnp.int32),
        pltpu.VMEM((EDGES_PER_TILE,), jnp.int32),
        pltpu.VMEM((EDGES_PER_TILE,), jnp.float32),
        pltpu.VMEM((N_PAD,), jnp.float32),
    ],
)
def _edge_scalar_stage(t_hbm, d_hbm, c_hbm, src_hbm, dst_hbm,
                       ex_hbm, sall_hbm,
                       t_v, d_v, c_v, src_v, dst_v, ex_v, s_v):
    wid = lax.axis_index("s") * 2 + lax.axis_index("c")
    base = wid * EDGES_PER_TILE
    pltpu.sync_copy(t_hbm, t_v)
    pltpu.sync_copy(d_hbm, d_v)
    pltpu.sync_copy(c_hbm, c_v)
    pltpu.sync_copy(src_hbm.at[pl.ds(base, EDGES_PER_TILE)], src_v)
    pltpu.sync_copy(dst_hbm.at[pl.ds(base, EDGES_PER_TILE)], dst_v)
    zero16 = jnp.zeros((16,), jnp.float32)

    def _zero(i, carry):
        s_v[pl.ds(i * 16, 16)] = zero16
        return carry

    lax.fori_loop(0, N_PAD // 16, _zero, 0)
    cvec = c_v[...]

    def _edges(j, carry):
        sl = pl.ds(j * 16, 16)
        s16 = src_v[sl]
        d16 = dst_v[sl]
        tg = plsc.load_gather(t_v, [s16])
        dg = plsc.load_gather(d_v, [d16])
        x = tg + dg
        e = jnp.maximum(x, 0.2 * x)
        exv = jnp.exp(e - cvec)
        ex_v[sl] = exv
        plsc.addupdate_scatter(s_v, [d16], exv)
        return carry

    lax.fori_loop(0, EDGES_PER_TILE // 16, _edges, 0)
    pltpu.sync_copy(ex_v, ex_hbm.at[pl.ds(base, EDGES_PER_TILE)])
    pltpu.sync_copy(s_v, sall_hbm.at[wid])


_COLS_PER_TILE = N_PAD // 32


@functools.partial(
    pl.kernel,
    out_type=jax.ShapeDtypeStruct((N_PAD,), jnp.float32),
    mesh=_SC_MESH,
    compiler_params=pltpu.CompilerParams(needs_layout_passes=False, use_tc_tiling_on_sc=False),
    scratch_types=[
        pltpu.VMEM((32, _COLS_PER_TILE), jnp.float32),
        pltpu.VMEM((_COLS_PER_TILE,), jnp.float32),
    ],
)
def _combine_stage(sall_hbm, rec_hbm, buf_v, acc_v):
    wid = lax.axis_index("s") * 2 + lax.axis_index("c")
    col0 = wid * _COLS_PER_TILE
    pltpu.sync_copy(sall_hbm.at[:, pl.ds(col0, _COLS_PER_TILE)], buf_v)
    nchunk = _COLS_PER_TILE // 16
    zero16 = jnp.zeros((16,), jnp.float32)

    def _zero(i, carry):
        acc_v[pl.ds(i * 16, 16)] = zero16
        return carry

    lax.fori_loop(0, nchunk, _zero, 0)

    def _add_row(c, carry):
        for k in range(nchunk):
            sl = pl.ds(k * 16, 16)
            acc_v[sl] = acc_v[sl] + buf_v[c, sl]
        return carry

    lax.fori_loop(0, 32, _add_row, 0)
    for k in range(nchunk):
        sl = pl.ds(k * 16, 16)
        acc_v[sl] = 1.0 / (acc_v[sl] + 1e-9)
    pltpu.sync_copy(acc_v, rec_hbm.at[pl.ds(col0, _COLS_PER_TILE)])


_Q = 640
_ACC_ROWS = _Q + 16
_BATCH = 32
_RING = 3
_CHUNK = 2048
_EPT_D = E_PAD // 16
_CSIZE = 2080


@functools.partial(
    pl.kernel,
    out_type=jax.ShapeDtypeStruct((N_PAD, 768), jnp.float32),
    mesh=_SC_MESH,
    compiler_params=pltpu.CompilerParams(needs_layout_passes=False, use_tc_tiling_on_sc=False),
    scratch_types=[
        pltpu.VMEM((_CHUNK,), jnp.int32),
        pltpu.VMEM((_CHUNK,), jnp.int32),
        pltpu.VMEM((_CHUNK,), jnp.float32),
        pltpu.VMEM((_CSIZE,), jnp.int32),
        pltpu.VMEM((_CSIZE,), jnp.int32),
        pltpu.VMEM((_CSIZE,), jnp.float32),
        [pltpu.VMEM((_BATCH,), jnp.int32) for _ in range(_RING)],
        [pltpu.VMEM((_BATCH,), jnp.int32) for _ in range(_RING)],
        [pltpu.VMEM((_BATCH,), jnp.float32) for _ in range(_RING)],
        [pltpu.VMEM((_BATCH, 768), jnp.float32) for _ in range(_RING)],
        pltpu.VMEM((16,), jnp.float32),
        [pltpu.SemaphoreType.DMA for _ in range(_RING)],
        [pltpu.SemaphoreType.DMA for _ in range(_RING)],
        pltpu.VMEM_SHARED((_ACC_ROWS, 768), jnp.float32),
    ],
)
def _aggregate_stage(h_hbm, src_hbm, dst_hbm, ex_hbm, rec_hbm, out_hbm,
                     srcc, dstc, exc, csrc, cdst, cex,
                     bsrc, bdst, bex, rowbuf, recb, gsem, ssem, acc_sh):
    core = lax.axis_index("c")
    sub = lax.axis_index("s")
    zero16f = jnp.zeros((16,), jnp.float32)
    zero16i = jnp.zeros((16,), jnp.int32)
    junk16 = jnp.full((16,), _Q, jnp.int32)
    ebase = sub * _EPT_D

    def _prep(p, b):
        for k in range(_BATCH // 16):
            sl = pl.ds(k * 16, 16)
            slc = pl.ds(b * _BATCH + k * 16, 16)
            bsrc[p][sl] = csrc[slc]
            bdst[p][sl] = cdst[slc]
            bex[p][sl] = cex[slc]

    def _scale(p):
        @plsc.parallel_loop(0, _BATCH, unroll=1)
        def _srow(i):
            av = plsc.load_gather(bex[p], [zero16i + i])
            for k in range(48):
                sl2 = pl.ds(k * 16, 16)
                rowbuf[p][i, sl2] = rowbuf[p][i, sl2] * av

    def _round(r, rcarry):
        sector = 8 * core + r
        base = sector * _Q

        @plsc.parallel_loop(0, _BATCH, unroll=1)
        def _zrow(i):
            for k in range(48):
                rowbuf[0][i, pl.ds(k * 16, 16)] = zero16f
        r0 = sub * (_ACC_ROWS // 16)
        pltpu.sync_copy(rowbuf[0], acc_sh.at[pl.ds(r0, 32)])
        pltpu.sync_copy(rowbuf[0].at[pl.ds(0, 9)],
                        acc_sh.at[pl.ds(r0 + 32, 9)])
        plsc.subcore_barrier()

        def _chunk(ch, off_in):
            e0 = ebase + ch * _CHUNK
            pltpu.sync_copy(src_hbm.at[pl.ds(e0, _CHUNK)], srcc)
            pltpu.sync_copy(dst_hbm.at[pl.ds(e0, _CHUNK)], dstc)
            pltpu.sync_copy(ex_hbm.at[pl.ds(e0, _CHUNK)], exc)

            def _grp(g, off):
                sl = pl.ds(g * 16, 16)
                rel = dstc[sl] - base
                m = (rel >= 0) & (rel < _Q)
                plsc.store_compressed(csrc.at[pl.ds(off, 16)], srcc[sl], mask=m)
                plsc.store_compressed(cdst.at[pl.ds(off, 16)], rel, mask=m)
                plsc.store_compressed(cex.at[pl.ds(off, 16)], exc[sl], mask=m)
                return off + jnp.sum(m.astype(jnp.int32))

            return lax.fori_loop(0, _CHUNK // 16, _grp, off_in)

        n_sel = lax.fori_loop(0, _EPT_D // _CHUNK, _chunk, 0)
        for k in range(224 // 16):
            sl = pl.ds(n_sel + k * 16, 16)
            csrc[sl] = zero16i
            cdst[sl] = junk16
            cex[sl] = zero16f

        for p in range(_RING):
            _prep(p, p)
            pltpu.async_copy(h_hbm.at[bsrc[p]], rowbuf[p], gsem[p])
        nb_total = (n_sel + _BATCH - 1) // _BATCH
        nbt = jnp.maximum((nb_total + _RING - 1) // _RING, 1)

        def _iter(t, carry):
            for p in range(_RING):
                pltpu.make_async_copy(h_hbm.at[bsrc[p]], rowbuf[p],
                                      gsem[p]).wait()
                _scale(p)
                pltpu.async_copy(rowbuf[p], acc_sh.at[bdst[p]], ssem[p],
                                 add=True)
            for p in range(_RING):
                pltpu.make_async_copy(rowbuf[p], acc_sh.at[bdst[p]],
                                      ssem[p]).wait()
                _prep(p, t * _RING + _RING + p)
                pltpu.async_copy(h_hbm.at[bsrc[p]], rowbuf[p], gsem[p])
            return carry

        lax.fori_loop(0, nbt, _iter, 0)
        for p in range(_RING):
            pltpu.make_async_copy(h_hbm.at[bsrc[p]], rowbuf[p], gsem[p]).wait()
        plsc.subcore_barrier()

        w0 = sub * (_Q // 16)
        for part0 in (0, 16, 24):
            gr0 = base + w0 + part0
            pltpu.sync_copy(acc_sh.at[pl.ds(w0 + part0, 16)],
                            rowbuf[0].at[pl.ds(0, 16)])
            pltpu.sync_copy(rec_hbm.at[pl.ds(gr0, 16)], recb)

            @plsc.parallel_loop(0, 16, unroll=1)
            def _erow(i):
                rv = plsc.load_gather(recb, [zero16i + i])
                for k in range(48):
                    sl2 = pl.ds(k * 16, 16)
                    x = rowbuf[0][i, sl2] * rv
                    rowbuf[0][i, sl2] = jnp.where(
                        x > 0.0, x, jnp.exp(jnp.minimum(x, 0.0)) - 1.0)
            pltpu.sync_copy(rowbuf[0].at[pl.ds(0, 16)],
                            out_hbm.at[pl.ds(gr0, 16)])
        plsc.subcore_barrier()
        return rcarry

    lax.fori_loop(0, 8, _round, 0)


def kernel(x_sent, x_type, W_sent, W_type, a_src, a_dst, edge_index):
    n_sent, _ = x_sent.shape
    n_type, _ = x_type.shape
    x_sent_p = jnp.pad(x_sent, ((0, N_PAD - n_sent), (0, 0)))
    x_type_p = jnp.pad(x_type, ((0, N_PAD - n_type), (0, 0)))
    a_src128 = jnp.broadcast_to(a_src[:, None], (768, 128))
    a_dst128 = jnp.broadcast_to(a_dst[:, None], (768, 128))

    h_type_p, t128, d128, c11 = _dense_stage(
        x_type_p, x_sent_p, W_type.T, W_sent.T, a_src128, a_dst128)
    t = t128[:, 0]
    d = d128[:, 0]
    C = c11[0, 0]

    src = edge_index[0]
    dst = edge_index[1]
    n_e = src.shape[0]
    per_tile_pad = (E_PAD - n_e) // 16
    src_p = jnp.concatenate(
        [src.reshape(16, n_e // 16),
         jnp.zeros((16, per_tile_pad), jnp.int32)], axis=1).reshape(-1)
    dst_p = jnp.concatenate(
        [dst.reshape(16, n_e // 16),
         jnp.full((16, per_tile_pad), n_sent, jnp.int32)], axis=1).reshape(-1)
    c16 = jnp.broadcast_to(C, (16,))

    ex_p, s_all = _edge_scalar_stage(t, d, c16, src_p, dst_p)
    rec = _combine_stage(s_all)
    out_p = _aggregate_stage(h_type_p, src_p, dst_p, ex_p, rec)
    return out_p[:n_sent]


# speedup vs baseline: 2.5399x; 1.6442x over previous
import functools

import jax
import jax.numpy as jnp
from jax import lax
from jax.experimental import pallas as pl
from jax.experimental.pallas import tpu as pltpu
from jax.experimental.pallas import tpu_sc as plsc

N_PAD = 10240
ROW_BLK = 512
E_PAD = 163840
EDGES_PER_TILE = E_PAD // 32


def _dense_body(x_type_ref, x_sent_ref, wt_type_ref, wt_sent_ref,
                a_src_ref, a_dst_ref, h_type_ref, t_ref, d_ref, c_ref,
                acc_ref):
    i = pl.program_id(0)
    nsteps = pl.num_programs(0)
    xt = x_type_ref[...]
    xs = x_sent_ref[...]
    wt = wt_type_ref[...]
    ws = wt_sent_ref[...]
    h_type = jnp.dot(xt, wt, preferred_element_type=jnp.float32)
    h_type_ref[...] = h_type
    t_blk = jnp.dot(h_type, a_src_ref[...], preferred_element_type=jnp.float32)
    t_ref[...] = t_blk
    vs = jnp.dot(ws, a_dst_ref[...], preferred_element_type=jnp.float32)
    d_blk = jnp.dot(xs, vs, preferred_element_type=jnp.float32)
    d_ref[...] = d_blk

    @pl.when(i == 0)
    def _():
        acc_ref[0] = -jnp.inf
        acc_ref[1] = -jnp.inf

    acc_ref[0] = jnp.maximum(acc_ref[0], jnp.max(t_blk))
    acc_ref[1] = jnp.maximum(acc_ref[1], jnp.max(d_blk))

    @pl.when(i == nsteps - 1)
    def _():
        m = acc_ref[0] + acc_ref[1]
        c_ref[0, 0] = jnp.maximum(m, 0.2 * m)


@functools.partial(jax.jit, static_argnames=())
def _dense_stage(x_type_p, x_sent_p, wt_type, wt_sent, a_src128, a_dst128):
    nblk = N_PAD // ROW_BLK
    return pl.pallas_call(
        _dense_body,
        grid=(nblk,),
        in_specs=[
            pl.BlockSpec((ROW_BLK, 512), lambda i: (i, 0)),
            pl.BlockSpec((ROW_BLK, 512), lambda i: (i, 0)),
            pl.BlockSpec((512, 768), lambda i: (0, 0)),
            pl.BlockSpec((512, 768), lambda i: (0, 0)),
            pl.BlockSpec((768, 128), lambda i: (0, 0)),
            pl.BlockSpec((768, 128), lambda i: (0, 0)),
        ],
        out_specs=[
            pl.BlockSpec((ROW_BLK, 768), lambda i: (i, 0)),
            pl.BlockSpec((ROW_BLK, 128), lambda i: (i, 0)),
            pl.BlockSpec((ROW_BLK, 128), lambda i: (i, 0)),
            pl.BlockSpec(memory_space=pltpu.SMEM),
        ],
        out_shape=[
            jax.ShapeDtypeStruct((N_PAD, 768), jnp.float32),
            jax.ShapeDtypeStruct((N_PAD, 128), jnp.float32),
            jax.ShapeDtypeStruct((N_PAD, 128), jnp.float32),
            jax.ShapeDtypeStruct((1, 1), jnp.float32),
        ],
        scratch_shapes=[pltpu.SMEM((2,), jnp.float32)],
    )(x_type_p, x_sent_p, wt_type, wt_sent, a_src128, a_dst128)


_SC_MESH = plsc.VectorSubcoreMesh(core_axis_name="c", subcore_axis_name="s")


@functools.partial(
    pl.kernel,
    out_type=[
        jax.ShapeDtypeStruct((E_PAD,), jnp.float32),
        jax.ShapeDtypeStruct((32, N_PAD), jnp.float32),
    ],
    mesh=_SC_MESH,
    compiler_params=pltpu.CompilerParams(needs_layout_passes=False, use_tc_tiling_on_sc=False),
    scratch_types=[
        pltpu.VMEM((N_PAD,), jnp.float32),
        pltpu.VMEM((N_PAD,), jnp.float32),
        pltpu.VMEM((16,), jnp.float32),
        pltpu.VMEM((EDGES_PER_TILE,), jnp.int32),
        pltpu.VMEM((EDGES_PER_TILE,), jnp.int32),
        pltpu.VMEM((EDGES_PER_TILE,), jnp.float32),
        pltpu.VMEM((N_PAD,), jnp.float32),
    ],
)
def _edge_scalar_stage(t_hbm, d_hbm, c_hbm, src_hbm, dst_hbm,
                       ex_hbm, sall_hbm,
                       t_v, d_v, c_v, src_v, dst_v, ex_v, s_v):
    wid = lax.axis_index("s") * 2 + lax.axis_index("c")
    base = wid * EDGES_PER_TILE
    pltpu.sync_copy(t_hbm, t_v)
    pltpu.sync_copy(d_hbm, d_v)
    pltpu.sync_copy(c_hbm, c_v)
    pltpu.sync_copy(src_hbm.at[pl.ds(base, EDGES_PER_TILE)], src_v)
    pltpu.sync_copy(dst_hbm.at[pl.ds(base, EDGES_PER_TILE)], dst_v)
    zero16 = jnp.zeros((16,), jnp.float32)

    def _zero(i, carry):
        s_v[pl.ds(i * 16, 16)] = zero16
        return carry

    lax.fori_loop(0, N_PAD // 16, _zero, 0)
    cvec = c_v[...]

    def _edges(j, carry):
        sl = pl.ds(j * 16, 16)
        s16 = src_v[sl]
        d16 = dst_v[sl]
        tg = plsc.load_gather(t_v, [s16])
        dg = plsc.load_gather(d_v, [d16])
        x = tg + dg
        e = jnp.maximum(x, 0.2 * x)
        exv = jnp.exp(e - cvec)
        ex_v[sl] = exv
        plsc.addupdate_scatter(s_v, [d16], exv)
        return carry

    lax.fori_loop(0, EDGES_PER_TILE // 16, _edges, 0)
    pltpu.sync_copy(ex_v, ex_hbm.at[pl.ds(base, EDGES_PER_TILE)])
    pltpu.sync_copy(s_v, sall_hbm.at[wid])


_COLS_PER_TILE = N_PAD // 32


@functools.partial(
    pl.kernel,
    out_type=jax.ShapeDtypeStruct((N_PAD,), jnp.float32),
    mesh=_SC_MESH,
    compiler_params=pltpu.CompilerParams(needs_layout_passes=False, use_tc_tiling_on_sc=False),
    scratch_types=[
        pltpu.VMEM((32, _COLS_PER_TILE), jnp.float32),
        pltpu.VMEM((_COLS_PER_TILE,), jnp.float32),
    ],
)
def _combine_stage(sall_hbm, rec_hbm, buf_v, acc_v):
    wid = lax.axis_index("s") * 2 + lax.axis_index("c")
    col0 = wid * _COLS_PER_TILE
    pltpu.sync_copy(sall_hbm.at[:, pl.ds(col0, _COLS_PER_TILE)], buf_v)
    nchunk = _COLS_PER_TILE // 16
    zero16 = jnp.zeros((16,), jnp.float32)

    def _zero(i, carry):
        acc_v[pl.ds(i * 16, 16)] = zero16
        return carry

    lax.fori_loop(0, nchunk, _zero, 0)

    def _add_row(c, carry):
        for k in range(nchunk):
            sl = pl.ds(k * 16, 16)
            acc_v[sl] = acc_v[sl] + buf_v[c, sl]
        return carry

    lax.fori_loop(0, 32, _add_row, 0)
    for k in range(nchunk):
        sl = pl.ds(k * 16, 16)
        acc_v[sl] = 1.0 / (acc_v[sl] + 1e-9)
    pltpu.sync_copy(acc_v, rec_hbm.at[pl.ds(col0, _COLS_PER_TILE)])


_Q = 1280
_ACC_ROWS = _Q + 16
_BATCH = 32
_RING = 2
_CHUNK = 1024
_EPT_D = E_PAD // 16
_CSIZE = 2112


@functools.partial(
    pl.kernel,
    out_type=jax.ShapeDtypeStruct((N_PAD, 768), jnp.float32),
    mesh=_SC_MESH,
    compiler_params=pltpu.CompilerParams(needs_layout_passes=False, use_tc_tiling_on_sc=False),
    scratch_types=[
        pltpu.VMEM((_CHUNK,), jnp.int32),
        pltpu.VMEM((_CHUNK,), jnp.int32),
        pltpu.VMEM((_CHUNK,), jnp.float32),
        pltpu.VMEM((_CSIZE,), jnp.int32),
        pltpu.VMEM((_CSIZE,), jnp.int32),
        pltpu.VMEM((_CSIZE,), jnp.float32),
        [pltpu.VMEM((_BATCH,), jnp.int32) for _ in range(_RING)],
        [pltpu.VMEM((_BATCH,), jnp.int32) for _ in range(_RING)],
        [pltpu.VMEM((_BATCH,), jnp.float32) for _ in range(_RING)],
        [pltpu.VMEM((_BATCH, 768), jnp.float32) for _ in range(_RING)],
        pltpu.VMEM((16,), jnp.float32),
        [pltpu.SemaphoreType.DMA for _ in range(_RING)],
        [pltpu.SemaphoreType.DMA for _ in range(_RING)],
        pltpu.VMEM_SHARED((_ACC_ROWS, 768), jnp.float32),
    ],
)
def _aggregate_stage(h_hbm, src_hbm, dst_hbm, ex_hbm, rec_hbm, out_hbm,
                     srcc, dstc, exc, csrc, cdst, cex,
                     bsrc, bdst, bex, rowbuf, recb, gsem, ssem, acc_sh):
    core = lax.axis_index("c")
    sub = lax.axis_index("s")
    zero16f = jnp.zeros((16,), jnp.float32)
    zero16i = jnp.zeros((16,), jnp.int32)
    junk16 = jnp.full((16,), _Q, jnp.int32)
    ebase = sub * _EPT_D

    def _prep(p, b):
        for k in range(_BATCH // 16):
            sl = pl.ds(k * 16, 16)
            slc = pl.ds(b * _BATCH + k * 16, 16)
            bsrc[p][sl] = csrc[slc]
            bdst[p][sl] = cdst[slc]
            bex[p][sl] = cex[slc]

    def _scale(p):
        @plsc.parallel_loop(0, _BATCH, unroll=1)
        def _srow(i):
            av = plsc.load_gather(bex[p], [zero16i + i])
            for k in range(48):
                sl2 = pl.ds(k * 16, 16)
                rowbuf[p][i, sl2] = rowbuf[p][i, sl2] * av

    def _round(r, rcarry):
        sector = 4 * core + r
        base = sector * _Q

        @plsc.parallel_loop(0, _BATCH, unroll=1)
        def _zrow(i):
            for k in range(48):
                rowbuf[0][i, pl.ds(k * 16, 16)] = zero16f
        r0 = sub * (_ACC_ROWS // 16)

        def _zpart(z, zcarry):
            pltpu.sync_copy(rowbuf[0], acc_sh.at[pl.ds(r0 + z * 32, 32)])
            return zcarry

        lax.fori_loop(0, 2, _zpart, 0)
        pltpu.sync_copy(rowbuf[0].at[pl.ds(0, 17)],
                        acc_sh.at[pl.ds(r0 + 64, 17)])
        plsc.subcore_barrier()

        def _chunk(ch, off_in):
            e0 = ebase + ch * _CHUNK
            pltpu.sync_copy(src_hbm.at[pl.ds(e0, _CHUNK)], srcc)
            pltpu.sync_copy(dst_hbm.at[pl.ds(e0, _CHUNK)], dstc)
            pltpu.sync_copy(ex_hbm.at[pl.ds(e0, _CHUNK)], exc)

            def _grp(g, off):
                sl = pl.ds(g * 16, 16)
                rel = dstc[sl] - base
                m = (rel >= 0) & (rel < _Q)
                plsc.store_compressed(csrc.at[pl.ds(off, 16)], srcc[sl], mask=m)
                plsc.store_compressed(cdst.at[pl.ds(off, 16)], rel, mask=m)
                plsc.store_compressed(cex.at[pl.ds(off, 16)], exc[sl], mask=m)
                return off + jnp.sum(m.astype(jnp.int32))

            return lax.fori_loop(0, _CHUNK // 16, _grp, off_in)

        n_sel = lax.fori_loop(0, _EPT_D // _CHUNK, _chunk, 0)
        for k in range(224 // 16):
            sl = pl.ds(n_sel + k * 16, 16)
            csrc[sl] = zero16i
            cdst[sl] = junk16
            cex[sl] = zero16f

        for p in range(_RING):
            _prep(p, p)
            pltpu.async_copy(h_hbm.at[bsrc[p]], rowbuf[p], gsem[p])
        nb_total = (n_sel + _BATCH - 1) // _BATCH
        nbt = jnp.maximum((nb_total + _RING - 1) // _RING, 1)

        def _iter(t, carry):
            for p in range(_RING):
                pltpu.make_async_copy(h_hbm.at[bsrc[p]], rowbuf[p],
                                      gsem[p]).wait()
                _scale(p)
                pltpu.async_copy(rowbuf[p], acc_sh.at[bdst[p]], ssem[p],
                                 add=True)
            for p in range(_RING):
                pltpu.make_async_copy(rowbuf[p], acc_sh.at[bdst[p]],
                                      ssem[p]).wait()
                _prep(p, t * _RING + _RING + p)
                pltpu.async_copy(h_hbm.at[bsrc[p]], rowbuf[p], gsem[p])
            return carry

        lax.fori_loop(0, nbt, _iter, 0)
        for p in range(_RING):
            pltpu.make_async_copy(h_hbm.at[bsrc[p]], rowbuf[p], gsem[p]).wait()
        plsc.subcore_barrier()

        w0 = sub * (_Q // 16)

        def _wbpart(wp, wcarry):
            part0 = wp * 16
            gr0 = base + w0 + part0
            pltpu.sync_copy(acc_sh.at[pl.ds(w0 + part0, 16)],
                            rowbuf[0].at[pl.ds(0, 16)])
            pltpu.sync_copy(rec_hbm.at[pl.ds(gr0, 16)], recb)

            @plsc.parallel_loop(0, 16, unroll=1)
            def _erow(i):
                rv = plsc.load_gather(recb, [zero16i + i])
                for k in range(48):
                    sl2 = pl.ds(k * 16, 16)
                    x = rowbuf[0][i, sl2] * rv
                    rowbuf[0][i, sl2] = jnp.where(
                        x > 0.0, x, jnp.exp(jnp.minimum(x, 0.0)) - 1.0)
            pltpu.sync_copy(rowbuf[0].at[pl.ds(0, 16)],
                            out_hbm.at[pl.ds(gr0, 16)])
            return wcarry

        lax.fori_loop(0, _Q // 256, _wbpart, 0)
        plsc.subcore_barrier()
        return rcarry

    lax.fori_loop(0, 4, _round, 0)


def kernel(x_sent, x_type, W_sent, W_type, a_src, a_dst, edge_index):
    n_sent, _ = x_sent.shape
    n_type, _ = x_type.shape
    x_sent_p = jnp.pad(x_sent, ((0, N_PAD - n_sent), (0, 0)))
    x_type_p = jnp.pad(x_type, ((0, N_PAD - n_type), (0, 0)))
    a_src128 = jnp.broadcast_to(a_src[:, None], (768, 128))
    a_dst128 = jnp.broadcast_to(a_dst[:, None], (768, 128))

    h_type_p, t128, d128, c11 = _dense_stage(
        x_type_p, x_sent_p, W_type.T, W_sent.T, a_src128, a_dst128)
    t = t128[:, 0]
    d = d128[:, 0]
    C = c11[0, 0]

    src = edge_index[0]
    dst = edge_index[1]
    n_e = src.shape[0]
    per_tile_pad = (E_PAD - n_e) // 16
    src_p = jnp.concatenate(
        [src.reshape(16, n_e // 16),
         jnp.zeros((16, per_tile_pad), jnp.int32)], axis=1).reshape(-1)
    dst_p = jnp.concatenate(
        [dst.reshape(16, n_e // 16),
         jnp.full((16, per_tile_pad), n_sent, jnp.int32)], axis=1).reshape(-1)
    c16 = jnp.broadcast_to(C, (16,))

    ex_p, s_all = _edge_scalar_stage(t, d, c16, src_p, dst_p)
    rec = _combine_stage(s_all)
    out_p = _aggregate_stage(h_type_p, src_p, dst_p, ex_p, rec)
    return out_p[:n_sent]
